# Initial kernel scaffold; baseline (speedup 1.0000x reference)
import jax
import jax.numpy as jnp
from jax.experimental import pallas as pl


def kernel(indices, tables, W1, b1, W2, b2, W3, b3):
    raise NotImplementedError("write your pallas kernel here")


# speedup vs baseline: 6.2328x; 6.2328x over previous
import functools

import jax
import jax.numpy as jnp
from jax.experimental import pallas as pl
from jax.experimental.pallas import tpu as pltpu
from jax.experimental.pallas import tpu_sc as plsc

_NC, _NS = 2, 16
_NW = _NC * _NS


def _sc_gather(src, srow_idx, n_rows, width):
    window = 256
    assert n_rows % (_NW * window) == 0
    b_per_w = n_rows // _NW
    n_steps = b_per_w // window
    mesh = plsc.VectorSubcoreMesh(core_axis_name="c", subcore_axis_name="s")

    @functools.partial(
        pl.kernel,
        out_type=jax.ShapeDtypeStruct((n_rows, width), src.dtype),
        mesh=mesh,
        scratch_types=[
            pltpu.VMEM((window,), jnp.int32),
            pltpu.VMEM((window, width), src.dtype),
            pltpu.SemaphoreType.DMA,
        ],
    )
    def gather_kernel(tab_hbm, idx_hbm, out_hbm, idx_v, rows_v, sem):
        wid = jax.lax.axis_index("s") * _NC + jax.lax.axis_index("c")
        base = wid * b_per_w

        @pl.loop(0, n_steps)
        def _(w):
            start = base + w * window
            pltpu.sync_copy(idx_hbm.at[pl.ds(start, window)], idx_v)
            pltpu.async_copy(tab_hbm.at[idx_v], rows_v, sem).wait()
            pltpu.sync_copy(rows_v, out_hbm.at[pl.ds(start, window)])

    return gather_kernel(src, srow_idx)


def _make_mlp_body(n_fields, groups):
    def _mlp_body(x4_ref, g_ref, w1_ref, b1_ref, w2_ref, b2_ref, w3_ref,
                  b3_ref, o_ref):
        bb = x4_ref.shape[0]
        lane_group = jax.lax.broadcasted_iota(jnp.int32, (bb, 128), 1) // 32
        pieces = []
        for f in range(n_fields):
            x4f = x4_ref[:, f * 128:(f + 1) * 128]
            gf = g_ref[:, f:f + 1]
            pieces.append(jnp.where(lane_group == gf, x4f, 0.0))
        xm = jnp.concatenate(pieces, axis=1).astype(jnp.bfloat16)
        h = jnp.dot(xm, w1_ref[...], preferred_element_type=jnp.float32)
        h = jnp.maximum(h + b1_ref[...], 0.0).astype(jnp.bfloat16)
        h = jnp.dot(h, w2_ref[...], preferred_element_type=jnp.float32)
        h = jnp.maximum(h + b2_ref[...], 0.0).astype(jnp.bfloat16)
        z = jnp.dot(h, w3_ref[...], preferred_element_type=jnp.float32)
        o_ref[...] = jax.nn.sigmoid(z + b3_ref[...])

    return _mlp_body


def _tc_mlp(x4, g, W1big, b1, W2, b2, W3, b3, block_b=512):
    bsz, kw = x4.shape
    n_fields = g.shape[1]
    hdim = W2.shape[0]
    assert bsz % block_b == 0
    grid = (bsz // block_b,)
    full = lambda shape: pl.BlockSpec(shape, lambda i: (0,) * len(shape))
    return pl.pallas_call(
        _make_mlp_body(n_fields, 4),
        grid=grid,
        in_specs=[
            pl.BlockSpec((block_b, kw), lambda i: (i, 0)),
            pl.BlockSpec((block_b, n_fields), lambda i: (i, 0)),
            full((kw, hdim)),
            full((1, hdim)),
            full((hdim, hdim)),
            full((1, hdim)),
            full((hdim, 1)),
            full((1, 1)),
        ],
        out_specs=pl.BlockSpec((block_b, 1), lambda i: (i, 0)),
        out_shape=jax.ShapeDtypeStruct((bsz, 1), jnp.float32),
    )(x4, g, W1big, b1, W2, b2, W3, b3)


def kernel(indices, tables, W1, b1, W2, b2, W3, b3):
    B, F = indices.shape
    _, V, D = tables.shape
    H = W1.shape[1]
    src = tables.reshape(F * V // 4, 4 * D)
    offsets = (jnp.arange(F, dtype=jnp.int32) * V)[None, :]
    flat_idx = indices + offsets
    srow = (flat_idx // 4).reshape(B * F)
    g = flat_idx % 4
    x4 = _sc_gather(src, srow, B * F, 4 * D)
    x4 = x4.reshape(B, F * 4 * D)
    W1big = jnp.tile(W1.reshape(F, 1, D, H), (1, 4, 1, 1)).reshape(F * 4 * D, H)
    return _tc_mlp(
        x4,
        g,
        W1big.astype(jnp.bfloat16),
        b1.reshape(1, -1),
        W2.astype(jnp.bfloat16),
        b2.reshape(1, -1),
        W3.astype(jnp.bfloat16),
        b3.reshape(1, 1),
    )


# speedup vs baseline: 14.2356x; 2.2840x over previous
import functools

import jax
import jax.numpy as jnp
from jax.experimental import pallas as pl
from jax.experimental.pallas import tpu as pltpu
from jax.experimental.pallas import tpu_sc as plsc

_NC, _NS = 2, 16
_NW = _NC * _NS


def _sc_gather(tables, row_idx, n_rows, n_flat, d):
    window = 32
    assert n_rows % (_NW * window * 2) == 0
    b_per_w = n_rows // _NW
    n_pairs = b_per_w // (2 * window)
    mesh = plsc.VectorSubcoreMesh(core_axis_name="c", subcore_axis_name="s")

    @functools.partial(
        pl.kernel,
        out_type=jax.ShapeDtypeStruct((n_rows, 1, d), tables.dtype),
        mesh=mesh,
        scratch_types=[
            pltpu.VMEM((window,), jnp.int32),
            pltpu.VMEM((window,), jnp.int32),
            pltpu.VMEM((window, 1, d), tables.dtype),
            pltpu.VMEM((window, 1, d), tables.dtype),
            pltpu.SemaphoreType.DMA,
            pltpu.SemaphoreType.DMA,
        ],
    )
    def gather_kernel(tab_hbm, idx_hbm, out_hbm, idx_v0, idx_v1, rows_v0,
                      rows_v1, sem0, sem1):
        tab = tab_hbm
        wid = jax.lax.axis_index("s") * _NC + jax.lax.axis_index("c")
        base = wid * b_per_w

        @pl.loop(0, n_pairs)
        def _(t):
            s0 = base + (2 * t) * window
            s1 = base + (2 * t + 1) * window
            pltpu.sync_copy(idx_hbm.at[pl.ds(s0, window)], idx_v0)
            cp0 = pltpu.async_copy(tab.at[idx_v0], rows_v0, sem0)
            pltpu.sync_copy(idx_hbm.at[pl.ds(s1, window)], idx_v1)
            cp1 = pltpu.async_copy(tab.at[idx_v1], rows_v1, sem1)
            cp0.wait()
            pltpu.sync_copy(rows_v0, out_hbm.at[pl.ds(s0, window)])
            cp1.wait()
            pltpu.sync_copy(rows_v1, out_hbm.at[pl.ds(s1, window)])

    return gather_kernel(tables, row_idx)


def _make_mlp_body(n_fields):
    def _mlp_body(x_ref, w1_ref, b1_ref, w2_ref, b2_ref, w3_ref, b3_ref,
                  o_ref):
        xm = jnp.concatenate(
            [x_ref[f] for f in range(n_fields)], axis=1
        ).astype(jnp.bfloat16)
        h = jnp.dot(xm, w1_ref[...], preferred_element_type=jnp.float32)
        h = jnp.maximum(h + b1_ref[...], 0.0).astype(jnp.bfloat16)
        h = jnp.dot(h, w2_ref[...], preferred_element_type=jnp.float32)
        h = jnp.maximum(h + b2_ref[...], 0.0).astype(jnp.bfloat16)
        z = jnp.dot(h, w3_ref[...], preferred_element_type=jnp.float32)
        o_ref[...] = jax.nn.sigmoid(z + b3_ref[...])

    return _mlp_body


def _tc_mlp(xfm, W1, b1, W2, b2, W3, b3, block_b=512):
    n_fields, bsz, d = xfm.shape
    hdim = W2.shape[0]
    assert bsz % block_b == 0
    grid = (bsz // block_b,)
    full = lambda shape: pl.BlockSpec(shape, lambda i: (0,) * len(shape))
    return pl.pallas_call(
        _make_mlp_body(n_fields),
        grid=grid,
        in_specs=[
            pl.BlockSpec((n_fields, block_b, d), lambda i: (0, i, 0)),
            full((n_fields * d, hdim)),
            full((1, hdim)),
            full((hdim, hdim)),
            full((1, hdim)),
            full((hdim, 1)),
            full((1, 1)),
        ],
        out_specs=pl.BlockSpec((block_b, 1), lambda i: (i, 0)),
        out_shape=jax.ShapeDtypeStruct((bsz, 1), jnp.float32),
    )(xfm, W1, b1, W2, b2, W3, b3)


def kernel(indices, tables, W1, b1, W2, b2, W3, b3):
    B, F = indices.shape
    _, V, D = tables.shape
    offsets = (jnp.arange(F, dtype=jnp.int32) * V)[None, :]
    flat_idx = ((indices + offsets).T).reshape(B * F)
    x = _sc_gather(tables.reshape(F * V, 1, D), flat_idx, B * F, F * V, D)
    xfm = x.reshape(F, B, D)
    return _tc_mlp(
        xfm,
        W1.astype(jnp.bfloat16),
        b1.reshape(1, -1),
        W2.astype(jnp.bfloat16),
        b2.reshape(1, -1),
        W3.astype(jnp.bfloat16),
        b3.reshape(1, 1),
    )


# speedup vs baseline: 21.3478x; 1.4996x over previous
import functools

import jax
import jax.numpy as jnp
from jax.experimental import pallas as pl
from jax.experimental.pallas import tpu as pltpu
from jax.experimental.pallas import tpu_sc as plsc

_NC, _NS = 2, 16
_NW = _NC * _NS


_WINDOW = 128
_NBUF = 4


def _sc_gather(tables, row_idx, n_rows, n_flat, d):
    window = _WINDOW
    assert n_rows % (_NW * window * _NBUF) == 0
    b_per_w = n_rows // _NW
    n_groups = b_per_w // (window * _NBUF)
    mesh = plsc.VectorSubcoreMesh(core_axis_name="c", subcore_axis_name="s")

    @functools.partial(
        pl.kernel,
        out_type=jax.ShapeDtypeStruct((n_rows, 1, d), tables.dtype),
        mesh=mesh,
        scratch_types=[
            pltpu.VMEM((b_per_w,), jnp.int32),
            *([pltpu.VMEM((window, 1, d), tables.dtype)] * _NBUF),
            *([pltpu.SemaphoreType.DMA] * _NBUF),
            pltpu.SemaphoreType.DMA,
        ],
    )
    def gather_kernel(tab_hbm, idx_hbm, out_hbm, idx_v, *bufs_sems):
        rows = bufs_sems[:_NBUF]
        sems = bufs_sems[_NBUF:2 * _NBUF]
        isem = bufs_sems[2 * _NBUF]
        tab = tab_hbm
        wid = jax.lax.axis_index("s") * _NC + jax.lax.axis_index("c")
        base = wid * b_per_w
        pltpu.async_copy(idx_hbm.at[pl.ds(base, b_per_w)], idx_v, isem).wait()

        @pl.loop(0, n_groups)
        def _(t):
            g0 = t * (window * _NBUF)
            for b in range(_NBUF):
                off = g0 + b * window
                pltpu.async_copy(
                    tab.at[idx_v.at[pl.ds(off, window)]], rows[b], sems[b]
                )
            for b in range(_NBUF):
                off = g0 + b * window
                pltpu.make_async_copy(
                    tab.at[idx_v.at[pl.ds(off, window)]], rows[b], sems[b]
                ).wait()
                pltpu.sync_copy(rows[b], out_hbm.at[pl.ds(base + off, window)])

    return gather_kernel(tables, row_idx)


def _make_mlp_body(n_fields):
    def _mlp_body(x_ref, w1_ref, b1_ref, w2_ref, b2_ref, w3_ref, b3_ref,
                  o_ref):
        xm = jnp.concatenate(
            [x_ref[f] for f in range(n_fields)], axis=1
        ).astype(jnp.bfloat16)
        h = jnp.dot(xm, w1_ref[...], preferred_element_type=jnp.float32)
        h = jnp.maximum(h + b1_ref[...], 0.0).astype(jnp.bfloat16)
        h = jnp.dot(h, w2_ref[...], preferred_element_type=jnp.float32)
        h = jnp.maximum(h + b2_ref[...], 0.0).astype(jnp.bfloat16)
        z = jnp.dot(h, w3_ref[...], preferred_element_type=jnp.float32)
        o_ref[...] = jax.nn.sigmoid(z + b3_ref[...])

    return _mlp_body


def _tc_mlp(xfm, W1, b1, W2, b2, W3, b3, block_b=512):
    n_fields, bsz, d = xfm.shape
    hdim = W2.shape[0]
    assert bsz % block_b == 0
    grid = (bsz // block_b,)
    full = lambda shape: pl.BlockSpec(shape, lambda i: (0,) * len(shape))
    return pl.pallas_call(
        _make_mlp_body(n_fields),
        grid=grid,
        in_specs=[
            pl.BlockSpec((n_fields, block_b, d), lambda i: (0, i, 0)),
            full((n_fields * d, hdim)),
            full((1, hdim)),
            full((hdim, hdim)),
            full((1, hdim)),
            full((hdim, 1)),
            full((1, 1)),
        ],
        out_specs=pl.BlockSpec((block_b, 1), lambda i: (i, 0)),
        out_shape=jax.ShapeDtypeStruct((bsz, 1), jnp.float32),
    )(xfm, W1, b1, W2, b2, W3, b3)


def kernel(indices, tables, W1, b1, W2, b2, W3, b3):
    B, F = indices.shape
    _, V, D = tables.shape
    offsets = (jnp.arange(F, dtype=jnp.int32) * V)[None, :]
    flat_idx = ((indices + offsets).T).reshape(B * F)
    x = _sc_gather(tables.reshape(F * V, 1, D), flat_idx, B * F, F * V, D)
    xfm = x.reshape(F, B, D)
    return _tc_mlp(
        xfm,
        W1.astype(jnp.bfloat16),
        b1.reshape(1, -1),
        W2.astype(jnp.bfloat16),
        b2.reshape(1, -1),
        W3.astype(jnp.bfloat16),
        b3.reshape(1, 1),
    )
